# Initial kernel scaffold; baseline (speedup 1.0000x reference)
import jax
import jax.numpy as jnp
from jax.experimental import pallas as pl


def kernel(x):
    raise NotImplementedError("write your pallas kernel here")


# speedup vs baseline: 1.5465x; 1.5465x over previous
import numpy as np
import jax
import jax.numpy as jnp
from jax import lax
from jax.experimental import pallas as pl
from jax.experimental.pallas import tpu as pltpu

_ANCH = np.array(
    [
        [1.3221, 1.73145],
        [3.19275, 4.00944],
        [5.05587, 8.09892],
        [9.47112, 4.84053],
        [11.2364, 10.0071],
    ],
    np.float32,
)
_G = 26
_NPIX = _G * _G
_NA = 5
_NCH = 85
_STRIDE = 32.0


def _body(x_ref, o_ref):
    a = pl.program_id(1)
    v = x_ref[0, 0]
    rows = lax.broadcasted_iota(jnp.int32, (_NCH, _NPIX), 0)
    p = lax.broadcasted_iota(jnp.int32, (_NCH, _NPIX), 1)
    s = 1.0 / (1.0 + jnp.exp(-v))
    e = jnp.exp(v)

    aw = _ANCH[0, 0] * jnp.float32(1.0)
    ah = _ANCH[0, 1] * jnp.float32(1.0)
    for k in range(1, _NA):
        aw = jnp.where(a == k, _ANCH[k, 0], aw)
        ah = jnp.where(a == k, _ANCH[k, 1], ah)
    aw32 = aw * _STRIDE
    ah32 = ah * _STRIDE

    gx32 = (p % _G).astype(jnp.float32) * _STRIDE
    gy32 = (p // _G).astype(jnp.float32) * _STRIDE
    add = jnp.where(rows == 0, gx32, jnp.where(rows == 1, gy32, 0.0))
    mult = jnp.where(
        rows < 2,
        _STRIDE,
        jnp.where(rows == 2, aw32, jnp.where(rows == 3, ah32, 1.0)),
    )
    y = jnp.where((rows == 2) | (rows == 3), e, s) * mult + add
    o_ref[0, 0] = y.T


def kernel(x):
    B = x.shape[0]
    x4 = x.reshape(B, _NA, _NCH, _NPIX)
    out = pl.pallas_call(
        _body,
        grid=(B, _NA),
        in_specs=[
            pl.BlockSpec((1, 1, _NCH, _NPIX), lambda b, a: (b, a, 0, 0)),
        ],
        out_specs=pl.BlockSpec((1, 1, _NPIX, _NCH), lambda b, a: (b, a, 0, 0)),
        out_shape=jax.ShapeDtypeStruct((B, _NA, _NPIX, _NCH), jnp.float32),
    )(x4)
    return out.reshape(B, _NA * _NPIX, _NCH)
